# baseline (speedup 1.0000x reference)
import jax
import jax.numpy as jnp
import numpy as np
from jax import lax
from jax.experimental import pallas as pl
from jax.experimental.pallas import tpu as pltpu
from jax.experimental.pallas import tpu_sc as plsc

_NOBLE_CORES = np.array([0, 2, 10, 18, 36, 54, 86, 118])


def _valence_count(z: int) -> float:
    if z <= 0:
        return 0.0
    return float(z - _NOBLE_CORES[_NOBLE_CORES < z].max())


_VALENCE_TABLE = np.zeros((128,), dtype=np.float32)
for _z in range(119):
    _VALENCE_TABLE[_z] = _valence_count(_z)

_BLK = 2560
_L = 16
_NC = 2
_NS = 16
_NW = _NC * _NS


def _wi_body(wt_ref, b_ref, emb_ref, wi_ref):
    ei = lax.dot_general(
        wt_ref[...], emb_ref[...],
        (((1,), (1,)), ((), ())),
        preferred_element_type=jnp.float32,
    )
    x = ei + b_ref[0, 0]
    wi = jnp.maximum(x, 0.0) + jnp.log1p(jnp.exp(-jnp.abs(x)))
    wi_ref[...] = wi.reshape(1, 1, _BLK)


def _compute_wi(embedding, W, b):
    n, d = embedding.shape
    nb = n // _BLK
    wt = W.reshape(1, d)
    b2 = b.reshape(1, 1)
    wi = pl.pallas_call(
        _wi_body,
        grid=(nb,),
        in_specs=[
            pl.BlockSpec((1, d), lambda i: (0, 0)),
            pl.BlockSpec((1, 1), lambda i: (0, 0)),
            pl.BlockSpec((_BLK, d), lambda i: (i, 0)),
        ],
        out_specs=pl.BlockSpec((1, 1, _BLK), lambda i: (i, 0, 0)),
        out_shape=jax.ShapeDtypeStruct((nb, 1, _BLK), jnp.float32),
    )(wt, b2, embedding)
    return wi.reshape(n)


def _sc_nel_body(ch, nsys, species_hbm, bi_hbm, tbl_hbm,
                 nel_hbm, nparts_hbm,
                 spec_v, bi_v, nel_v, tbl_v, nacc):
    wid = lax.axis_index("s") * _NC + lax.axis_index("c")
    base = wid * ch
    pltpu.sync_copy(tbl_hbm, tbl_v)
    pltpu.sync_copy(species_hbm.at[pl.ds(base, ch)], spec_v)
    pltpu.sync_copy(bi_hbm.at[pl.ds(base, ch)], bi_v)

    zeros = jnp.zeros((_L,), jnp.float32)

    def zero_body(j, c):
        nacc[pl.ds(j * _L, _L)] = zeros
        return c

    lax.fori_loop(0, nsys // _L, zero_body, 0)

    def body(i, c):
        sl = pl.ds(i * _L, _L)
        n16 = plsc.load_gather(tbl_v, [spec_v[sl]])
        nel_v[sl] = n16
        plsc.addupdate_scatter(nacc, [bi_v[sl]], n16)
        return c

    lax.fori_loop(0, ch // _L, body, 0)

    pltpu.sync_copy(nel_v, nel_hbm.at[pl.ds(base, ch)])
    pltpu.sync_copy(nacc, nparts_hbm.at[pl.ds(wid * nsys, nsys)])


def _sc_wi_body(ch, nsys, bi_hbm, wi_hbm, wparts_hbm,
                bi_v, wi_v, wacc):
    wid = lax.axis_index("s") * _NC + lax.axis_index("c")
    base = wid * ch
    pltpu.sync_copy(bi_hbm.at[pl.ds(base, ch)], bi_v)
    pltpu.sync_copy(wi_hbm.at[pl.ds(base, ch)], wi_v)

    zeros = jnp.zeros((_L,), jnp.float32)

    def zero_body(j, c):
        wacc[pl.ds(j * _L, _L)] = zeros
        return c

    lax.fori_loop(0, nsys // _L, zero_body, 0)

    def body(i, c):
        sl = pl.ds(i * _L, _L)
        plsc.addupdate_scatter(wacc, [bi_v[sl]], wi_v[sl])
        return c

    lax.fori_loop(0, ch // _L, body, 0)

    pltpu.sync_copy(wacc, wparts_hbm.at[pl.ds(wid * nsys, nsys)])


def _sc_final_body(ch, nsys, wparts_hbm, nparts_hbm, tc_hbm,
                   wi_hbm, nel_hbm, bi_hbm, q_hbm,
                   wp_v, np_v, tc_v, f_v, wi_v, nel_v, bi_v, q_v):
    wid = lax.axis_index("s") * _NC + lax.axis_index("c")
    base = wid * ch
    pltpu.sync_copy(wparts_hbm, wp_v)
    pltpu.sync_copy(nparts_hbm, np_v)
    pltpu.sync_copy(tc_hbm, tc_v)
    pltpu.sync_copy(wi_hbm.at[pl.ds(base, ch)], wi_v)
    pltpu.sync_copy(nel_hbm.at[pl.ds(base, ch)], nel_v)
    pltpu.sync_copy(bi_hbm.at[pl.ds(base, ch)], bi_v)

    zeros = jnp.zeros((_L,), jnp.float32)

    def fbody(cidx, c):
        def rbody(r, accs):
            aw, an = accs
            off = r * nsys + cidx * _L
            return (aw + wp_v[pl.ds(off, _L)], an + np_v[pl.ds(off, _L)])

        aw, an = lax.fori_loop(0, _NW, rbody, (zeros, zeros))
        f_v[pl.ds(cidx * _L, _L)] = (an - tc_v[pl.ds(cidx * _L, _L)]) / aw
        return c

    lax.fori_loop(0, nsys // _L, fbody, 0)

    def body(i, c):
        sl = pl.ds(i * _L, _L)
        fg = plsc.load_gather(f_v, [bi_v[sl]])
        q_v[sl] = nel_v[sl] - wi_v[sl] * fg
        return c

    lax.fori_loop(0, ch // _L, body, 0)
    pltpu.sync_copy(q_v, q_hbm.at[pl.ds(base, ch)])


def kernel(species, embedding, batch_index, natoms, total_charge, W, b):
    n = embedding.shape[0]
    nsys = natoms.shape[0]
    ch = n // _NW
    tbl = jnp.asarray(_VALENCE_TABLE)
    mesh = plsc.VectorSubcoreMesh(core_axis_name="c", subcore_axis_name="s")
    sc_params = pltpu.CompilerParams(
        needs_layout_passes=False, skip_device_barrier=True
    )

    def nel_body(*refs):
        _sc_nel_body(ch, nsys, *refs)

    nel, nparts = pl.kernel(
        nel_body,
        out_type=[
            jax.ShapeDtypeStruct((n,), jnp.float32),
            jax.ShapeDtypeStruct((_NW * nsys,), jnp.float32),
        ],
        mesh=mesh,
        compiler_params=sc_params,
        scratch_types=[
            pltpu.VMEM((ch,), jnp.int32),
            pltpu.VMEM((ch,), jnp.int32),
            pltpu.VMEM((ch,), jnp.float32),
            pltpu.VMEM((128,), jnp.float32),
            pltpu.VMEM((nsys,), jnp.float32),
        ],
    )(species, batch_index, tbl)

    wi = _compute_wi(embedding, W, b)

    def wi_body(*refs):
        _sc_wi_body(ch, nsys, *refs)

    wparts = pl.kernel(
        wi_body,
        out_type=jax.ShapeDtypeStruct((_NW * nsys,), jnp.float32),
        mesh=mesh,
        compiler_params=sc_params,
        scratch_types=[
            pltpu.VMEM((ch,), jnp.int32),
            pltpu.VMEM((ch,), jnp.float32),
            pltpu.VMEM((nsys,), jnp.float32),
        ],
    )(batch_index, wi)

    def final_body(*refs):
        _sc_final_body(ch, nsys, *refs)

    q = pl.kernel(
        final_body,
        out_type=jax.ShapeDtypeStruct((n,), jnp.float32),
        mesh=mesh,
        compiler_params=sc_params,
        scratch_types=[
            pltpu.VMEM((_NW * nsys,), jnp.float32),
            pltpu.VMEM((_NW * nsys,), jnp.float32),
            pltpu.VMEM((nsys,), jnp.float32),
            pltpu.VMEM((nsys,), jnp.float32),
            pltpu.VMEM((ch,), jnp.float32),
            pltpu.VMEM((ch,), jnp.float32),
            pltpu.VMEM((ch,), jnp.int32),
            pltpu.VMEM((ch,), jnp.float32),
        ],
    )(wparts, nparts, total_charge, wi, nel, batch_index)
    return q


# speedup vs baseline: 1.5249x; 1.5249x over previous
import jax
import jax.numpy as jnp
import numpy as np
from jax import lax
from jax.experimental import pallas as pl
from jax.experimental.pallas import tpu as pltpu
from jax.experimental.pallas import tpu_sc as plsc

_NOBLE_CORES = np.array([0, 2, 10, 18, 36, 54, 86, 118])


def _valence_count(z: int) -> float:
    if z <= 0:
        return 0.0
    return float(z - _NOBLE_CORES[_NOBLE_CORES < z].max())


_VALENCE_TABLE = np.zeros((128,), dtype=np.float32)
for _z in range(119):
    _VALENCE_TABLE[_z] = _valence_count(_z)

_BLK = 6400
_L = 16
_NC = 2
_NS = 16
_NW = _NC * _NS


def _wi_body(wt_ref, b_ref, emb_ref, wi_ref):
    ei = lax.dot_general(
        wt_ref[...], emb_ref[...],
        (((1,), (1,)), ((), ())),
        preferred_element_type=jnp.float32,
    )
    x = ei + b_ref[0, 0]
    wi = jnp.maximum(x, 0.0) + jnp.log1p(jnp.exp(-jnp.abs(x)))
    wi_ref[...] = wi.reshape(1, 1, _BLK)


def _compute_wi(embedding, W, b):
    n, d = embedding.shape
    nb = n // _BLK
    wt = W.reshape(1, d)
    b2 = b.reshape(1, 1)
    wi = pl.pallas_call(
        _wi_body,
        grid=(nb,),
        in_specs=[
            pl.BlockSpec((1, d), lambda i: (0, 0)),
            pl.BlockSpec((1, 1), lambda i: (0, 0)),
            pl.BlockSpec((_BLK, d), lambda i: (i, 0)),
        ],
        out_specs=pl.BlockSpec((1, 1, _BLK), lambda i: (i, 0, 0)),
        out_shape=jax.ShapeDtypeStruct((nb, 1, _BLK), jnp.float32),
    )(wt, b2, embedding)
    return wi.reshape(n)


def _sc_partials_body(ch, nsys, species_hbm, bi_hbm, wi_hbm, tbl_hbm,
                      nel_hbm, wparts_hbm, nparts_hbm,
                      spec_v, bi_v, wi_v, nel_v, tbl_v, wacc, nacc):
    wid = lax.axis_index("s") * _NC + lax.axis_index("c")
    base = wid * ch
    pltpu.sync_copy(tbl_hbm, tbl_v)
    pltpu.sync_copy(species_hbm.at[pl.ds(base, ch)], spec_v)
    pltpu.sync_copy(bi_hbm.at[pl.ds(base, ch)], bi_v)
    pltpu.sync_copy(wi_hbm.at[pl.ds(base, ch)], wi_v)

    zeros = jnp.zeros((_L,), jnp.float32)

    def zero_body(j, c):
        wacc[pl.ds(j * _L, _L)] = zeros
        nacc[pl.ds(j * _L, _L)] = zeros
        return c

    lax.fori_loop(0, nsys // _L, zero_body, 0)

    lane_off = lax.iota(jnp.int32, _L) * (ch // _L)

    def body(i, c):
        idx = lane_off + i
        sp = plsc.load_gather(spec_v, [idx])
        b16 = plsc.load_gather(bi_v, [idx])
        w16 = plsc.load_gather(wi_v, [idx])
        n16 = plsc.load_gather(tbl_v, [sp])
        plsc.store_scatter(nel_v, [idx], n16)
        plsc.addupdate_scatter(wacc, [b16], w16)
        plsc.addupdate_scatter(nacc, [b16], n16)
        return c

    lax.fori_loop(0, ch // _L, body, 0)

    pltpu.sync_copy(nel_v, nel_hbm.at[pl.ds(base, ch)])
    pltpu.sync_copy(wacc, wparts_hbm.at[pl.ds(wid * nsys, nsys)])
    pltpu.sync_copy(nacc, nparts_hbm.at[pl.ds(wid * nsys, nsys)])


def _sc_final_body(ch, nsys, wparts_hbm, nparts_hbm, tc_hbm,
                   wi_hbm, nel_hbm, bi_hbm, q_hbm,
                   wp_v, np_v, tc_v, f_v, wi_v, nel_v, bi_v, q_v):
    wid = lax.axis_index("s") * _NC + lax.axis_index("c")
    base = wid * ch
    pltpu.sync_copy(wparts_hbm, wp_v)
    pltpu.sync_copy(nparts_hbm, np_v)
    pltpu.sync_copy(tc_hbm, tc_v)
    pltpu.sync_copy(wi_hbm.at[pl.ds(base, ch)], wi_v)
    pltpu.sync_copy(nel_hbm.at[pl.ds(base, ch)], nel_v)
    pltpu.sync_copy(bi_hbm.at[pl.ds(base, ch)], bi_v)

    zeros = jnp.zeros((_L,), jnp.float32)

    def fbody(cidx, c):
        def rbody(r, accs):
            aw, an = accs
            off = r * nsys + cidx * _L
            return (aw + wp_v[pl.ds(off, _L)], an + np_v[pl.ds(off, _L)])

        aw, an = lax.fori_loop(0, _NW, rbody, (zeros, zeros))
        f_v[pl.ds(cidx * _L, _L)] = (an - tc_v[pl.ds(cidx * _L, _L)]) / aw
        return c

    lax.fori_loop(0, nsys // _L, fbody, 0)

    def body(i, c):
        sl = pl.ds(i * _L, _L)
        fg = plsc.load_gather(f_v, [bi_v[sl]])
        q_v[sl] = nel_v[sl] - wi_v[sl] * fg
        return c

    lax.fori_loop(0, ch // _L, body, 0)
    pltpu.sync_copy(q_v, q_hbm.at[pl.ds(base, ch)])


def kernel(species, embedding, batch_index, natoms, total_charge, W, b):
    n = embedding.shape[0]
    nsys = natoms.shape[0]
    ch = n // _NW
    wi = _compute_wi(embedding, W, b)
    tbl = jnp.asarray(_VALENCE_TABLE)
    mesh = plsc.VectorSubcoreMesh(core_axis_name="c", subcore_axis_name="s")
    sc_params = pltpu.CompilerParams(
        needs_layout_passes=False, skip_device_barrier=True
    )

    def partials_body(*refs):
        _sc_partials_body(ch, nsys, *refs)

    nel, wparts, nparts = pl.kernel(
        partials_body,
        out_type=[
            jax.ShapeDtypeStruct((n,), jnp.float32),
            jax.ShapeDtypeStruct((_NW * nsys,), jnp.float32),
            jax.ShapeDtypeStruct((_NW * nsys,), jnp.float32),
        ],
        mesh=mesh,
        compiler_params=sc_params,
        scratch_types=[
            pltpu.VMEM((ch,), jnp.int32),
            pltpu.VMEM((ch,), jnp.int32),
            pltpu.VMEM((ch,), jnp.float32),
            pltpu.VMEM((ch,), jnp.float32),
            pltpu.VMEM((128,), jnp.float32),
            pltpu.VMEM((nsys,), jnp.float32),
            pltpu.VMEM((nsys,), jnp.float32),
        ],
    )(species, batch_index, wi, tbl)

    def final_body(*refs):
        _sc_final_body(ch, nsys, *refs)

    q = pl.kernel(
        final_body,
        out_type=jax.ShapeDtypeStruct((n,), jnp.float32),
        mesh=mesh,
        compiler_params=sc_params,
        scratch_types=[
            pltpu.VMEM((_NW * nsys,), jnp.float32),
            pltpu.VMEM((_NW * nsys,), jnp.float32),
            pltpu.VMEM((nsys,), jnp.float32),
            pltpu.VMEM((nsys,), jnp.float32),
            pltpu.VMEM((ch,), jnp.float32),
            pltpu.VMEM((ch,), jnp.float32),
            pltpu.VMEM((ch,), jnp.int32),
            pltpu.VMEM((ch,), jnp.float32),
        ],
    )(wparts, nparts, total_charge, wi, nel, batch_index)
    return q


# speedup vs baseline: 1.7634x; 1.1564x over previous
import jax
import jax.numpy as jnp
import numpy as np
from jax import lax
from jax.experimental import pallas as pl
from jax.experimental.pallas import tpu as pltpu
from jax.experimental.pallas import tpu_sc as plsc

_NOBLE_CORES = np.array([0, 2, 10, 18, 36, 54, 86, 118])


def _valence_count(z: int) -> float:
    if z <= 0:
        return 0.0
    return float(z - _NOBLE_CORES[_NOBLE_CORES < z].max())


_VALENCE_TABLE = np.zeros((128,), dtype=np.float32)
for _z in range(119):
    _VALENCE_TABLE[_z] = _valence_count(_z)

_BLK = 12800
_L = 16
_NC = 2
_NS = 16
_NW = _NC * _NS


def _wi_body(wt_ref, b_ref, emb_ref, wi_ref):
    ei = lax.dot_general(
        wt_ref[...], emb_ref[...],
        (((1,), (1,)), ((), ())),
        preferred_element_type=jnp.float32,
    )
    x = ei + b_ref[0, 0]
    wi = jnp.maximum(x, 0.0) + jnp.log1p(jnp.exp(-jnp.abs(x)))
    wi_ref[...] = wi.reshape(1, 1, _BLK)


def _compute_wi(embedding, W, b):
    n, d = embedding.shape
    nb = n // _BLK
    wt = W.reshape(1, d)
    b2 = b.reshape(1, 1)
    wi = pl.pallas_call(
        _wi_body,
        grid=(nb,),
        in_specs=[
            pl.BlockSpec((1, d), lambda i: (0, 0)),
            pl.BlockSpec((1, 1), lambda i: (0, 0)),
            pl.BlockSpec((_BLK, d), lambda i: (i, 0)),
        ],
        out_specs=pl.BlockSpec((1, 1, _BLK), lambda i: (i, 0, 0)),
        out_shape=jax.ShapeDtypeStruct((nb, 1, _BLK), jnp.float32),
    )(wt, b2, embedding)
    return wi.reshape(n)


def _sc_partials_body(ch, nsys, species_hbm, bi_hbm, wi_hbm, tbl_hbm,
                      nel_hbm, wparts_hbm, nparts_hbm,
                      spec_v, bi_v, wi_v, nel_v, tbl_v, wacc, nacc):
    wid = lax.axis_index("s") * _NC + lax.axis_index("c")
    base = wid * ch
    pltpu.sync_copy(tbl_hbm, tbl_v)
    pltpu.sync_copy(species_hbm.at[pl.ds(base, ch)], spec_v)
    pltpu.sync_copy(bi_hbm.at[pl.ds(base, ch)], bi_v)
    pltpu.sync_copy(wi_hbm.at[pl.ds(base, ch)], wi_v)

    zeros = jnp.zeros((_L,), jnp.float32)

    def zero_body(j, c):
        wacc[pl.ds(j * _L, _L)] = zeros
        nacc[pl.ds(j * _L, _L)] = zeros
        return c

    lax.fori_loop(0, nsys // _L, zero_body, 0)

    lane_off = lax.iota(jnp.int32, _L) * (ch // _L)

    def body(i, c):
        idx = lane_off + i
        sp = plsc.load_gather(spec_v, [idx])
        b16 = plsc.load_gather(bi_v, [idx])
        w16 = plsc.load_gather(wi_v, [idx])
        n16 = plsc.load_gather(tbl_v, [sp])
        plsc.store_scatter(nel_v, [idx], n16)
        plsc.addupdate_scatter(wacc, [b16], w16)
        plsc.addupdate_scatter(nacc, [b16], n16)
        return c

    lax.fori_loop(0, ch // _L, body, 0)

    pltpu.sync_copy(nel_v, nel_hbm.at[pl.ds(base, ch)])
    pltpu.sync_copy(wacc, wparts_hbm.at[pl.ds(wid * nsys, nsys)])
    pltpu.sync_copy(nacc, nparts_hbm.at[pl.ds(wid * nsys, nsys)])


def _sc_final_body(ch, nsys, wparts_hbm, nparts_hbm, tc_hbm,
                   wi_hbm, nel_hbm, bi_hbm, q_hbm,
                   wp_v, np_v, tc_v, f_v, wi_v, nel_v, bi_v, q_v):
    wid = lax.axis_index("s") * _NC + lax.axis_index("c")
    base = wid * ch
    pltpu.sync_copy(wparts_hbm, wp_v)
    pltpu.sync_copy(nparts_hbm, np_v)
    pltpu.sync_copy(tc_hbm, tc_v)
    pltpu.sync_copy(wi_hbm.at[pl.ds(base, ch)], wi_v)
    pltpu.sync_copy(nel_hbm.at[pl.ds(base, ch)], nel_v)
    pltpu.sync_copy(bi_hbm.at[pl.ds(base, ch)], bi_v)

    zeros = jnp.zeros((_L,), jnp.float32)

    def fbody(cidx, c):
        aw, an = zeros, zeros
        for r in range(_NW):
            off = r * nsys + cidx * _L
            aw = aw + wp_v[pl.ds(off, _L)]
            an = an + np_v[pl.ds(off, _L)]
        f_v[pl.ds(cidx * _L, _L)] = (an - tc_v[pl.ds(cidx * _L, _L)]) / aw
        return c

    lax.fori_loop(0, nsys // _L, fbody, 0)

    def body(i, c):
        sl = pl.ds(i * _L, _L)
        fg = plsc.load_gather(f_v, [bi_v[sl]])
        q_v[sl] = nel_v[sl] - wi_v[sl] * fg
        return c

    lax.fori_loop(0, ch // _L, body, 0)
    pltpu.sync_copy(q_v, q_hbm.at[pl.ds(base, ch)])


def kernel(species, embedding, batch_index, natoms, total_charge, W, b):
    n = embedding.shape[0]
    nsys = natoms.shape[0]
    ch = n // _NW
    wi = _compute_wi(embedding, W, b)
    tbl = jnp.asarray(_VALENCE_TABLE)
    mesh = plsc.VectorSubcoreMesh(core_axis_name="c", subcore_axis_name="s")
    sc_params = pltpu.CompilerParams(
        needs_layout_passes=False, skip_device_barrier=True
    )

    def partials_body(*refs):
        _sc_partials_body(ch, nsys, *refs)

    nel, wparts, nparts = pl.kernel(
        partials_body,
        out_type=[
            jax.ShapeDtypeStruct((n,), jnp.float32),
            jax.ShapeDtypeStruct((_NW * nsys,), jnp.float32),
            jax.ShapeDtypeStruct((_NW * nsys,), jnp.float32),
        ],
        mesh=mesh,
        compiler_params=sc_params,
        scratch_types=[
            pltpu.VMEM((ch,), jnp.int32),
            pltpu.VMEM((ch,), jnp.int32),
            pltpu.VMEM((ch,), jnp.float32),
            pltpu.VMEM((ch,), jnp.float32),
            pltpu.VMEM((128,), jnp.float32),
            pltpu.VMEM((nsys,), jnp.float32),
            pltpu.VMEM((nsys,), jnp.float32),
        ],
    )(species, batch_index, wi, tbl)

    def final_body(*refs):
        _sc_final_body(ch, nsys, *refs)

    q = pl.kernel(
        final_body,
        out_type=jax.ShapeDtypeStruct((n,), jnp.float32),
        mesh=mesh,
        compiler_params=sc_params,
        scratch_types=[
            pltpu.VMEM((_NW * nsys,), jnp.float32),
            pltpu.VMEM((_NW * nsys,), jnp.float32),
            pltpu.VMEM((nsys,), jnp.float32),
            pltpu.VMEM((nsys,), jnp.float32),
            pltpu.VMEM((ch,), jnp.float32),
            pltpu.VMEM((ch,), jnp.float32),
            pltpu.VMEM((ch,), jnp.int32),
            pltpu.VMEM((ch,), jnp.float32),
        ],
    )(wparts, nparts, total_charge, wi, nel, batch_index)
    return q


# speedup vs baseline: 1.8386x; 1.0426x over previous
import jax
import jax.numpy as jnp
import numpy as np
from jax import lax
from jax.experimental import pallas as pl
from jax.experimental.pallas import tpu as pltpu
from jax.experimental.pallas import tpu_sc as plsc

_NOBLE_CORES = np.array([0, 2, 10, 18, 36, 54, 86, 118])


def _valence_count(z: int) -> float:
    if z <= 0:
        return 0.0
    return float(z - _NOBLE_CORES[_NOBLE_CORES < z].max())


_VALENCE_TABLE = np.zeros((128,), dtype=np.float32)
for _z in range(119):
    _VALENCE_TABLE[_z] = _valence_count(_z)

_BLK = 32000
_UNROLL = 5
_L = 16
_NC = 2
_NS = 16
_NW = _NC * _NS


def _wi_body(wt_ref, b_ref, emb_ref, wi_ref):
    ei = lax.dot_general(
        wt_ref[...], emb_ref[...],
        (((1,), (1,)), ((), ())),
        preferred_element_type=jnp.float32,
    )
    x = ei + b_ref[0, 0]
    wi = jnp.maximum(x, 0.0) + jnp.log1p(jnp.exp(-jnp.abs(x)))
    wi_ref[...] = wi.reshape(1, 1, _BLK)


def _compute_wi(embedding, W, b):
    n, d = embedding.shape
    nb = n // _BLK
    wt = W.reshape(1, d)
    b2 = b.reshape(1, 1)
    wi = pl.pallas_call(
        _wi_body,
        grid=(nb,),
        in_specs=[
            pl.BlockSpec((1, d), lambda i: (0, 0)),
            pl.BlockSpec((1, 1), lambda i: (0, 0)),
            pl.BlockSpec((_BLK, d), lambda i: (i, 0)),
        ],
        out_specs=pl.BlockSpec((1, 1, _BLK), lambda i: (i, 0, 0)),
        out_shape=jax.ShapeDtypeStruct((nb, 1, _BLK), jnp.float32),
    )(wt, b2, embedding)
    return wi.reshape(n)


def _sc_partials_body(ch, nsys, species_hbm, bi_hbm, wi_hbm, tbl_hbm,
                      nel_hbm, wparts_hbm, nparts_hbm,
                      spec_v, bi_v, wi_v, nel_v, tbl_v, wacc, nacc, sem):
    wid = lax.axis_index("s") * _NC + lax.axis_index("c")
    base = wid * ch
    sl_h = pl.ds(base, ch)
    copies = [
        pltpu.make_async_copy(tbl_hbm, tbl_v, sem),
        pltpu.make_async_copy(species_hbm.at[sl_h], spec_v, sem),
        pltpu.make_async_copy(bi_hbm.at[sl_h], bi_v, sem),
        pltpu.make_async_copy(wi_hbm.at[sl_h], wi_v, sem),
    ]
    for c in copies:
        c.start()

    zeros = jnp.zeros((_L,), jnp.float32)
    for j in range(nsys // _L):
        wacc[pl.ds(j * _L, _L)] = zeros
        nacc[pl.ds(j * _L, _L)] = zeros
    for c in copies:
        c.wait()

    lane_off = lax.iota(jnp.int32, _L) * (ch // _L)

    def body(i, c):
        for j in range(_UNROLL):
            idx = lane_off + i * _UNROLL + j
            sp = plsc.load_gather(spec_v, [idx])
            b16 = plsc.load_gather(bi_v, [idx])
            w16 = plsc.load_gather(wi_v, [idx])
            n16 = plsc.load_gather(tbl_v, [sp])
            plsc.store_scatter(nel_v, [idx], n16)
            plsc.addupdate_scatter(wacc, [b16], w16)
            plsc.addupdate_scatter(nacc, [b16], n16)
        return c

    lax.fori_loop(0, ch // (_L * _UNROLL), body, 0)

    pltpu.sync_copy(nel_v, nel_hbm.at[pl.ds(base, ch)])
    pltpu.sync_copy(wacc, wparts_hbm.at[pl.ds(wid * nsys, nsys)])
    pltpu.sync_copy(nacc, nparts_hbm.at[pl.ds(wid * nsys, nsys)])


def _sc_final_body(ch, nsys, wparts_hbm, nparts_hbm, tc_hbm,
                   wi_hbm, nel_hbm, bi_hbm, q_hbm,
                   wp_v, np_v, tc_v, f_v, wi_v, nel_v, bi_v, q_v, sem):
    wid = lax.axis_index("s") * _NC + lax.axis_index("c")
    base = wid * ch
    sl_h = pl.ds(base, ch)
    copies = [
        pltpu.make_async_copy(wparts_hbm, wp_v, sem),
        pltpu.make_async_copy(nparts_hbm, np_v, sem),
        pltpu.make_async_copy(tc_hbm, tc_v, sem),
        pltpu.make_async_copy(wi_hbm.at[sl_h], wi_v, sem),
        pltpu.make_async_copy(nel_hbm.at[sl_h], nel_v, sem),
        pltpu.make_async_copy(bi_hbm.at[sl_h], bi_v, sem),
    ]
    for c in copies:
        c.start()
    for c in copies:
        c.wait()

    zeros = jnp.zeros((_L,), jnp.float32)

    def fbody(cidx, c):
        aw, an = zeros, zeros
        for r in range(_NW):
            off = r * nsys + cidx * _L
            aw = aw + wp_v[pl.ds(off, _L)]
            an = an + np_v[pl.ds(off, _L)]
        f_v[pl.ds(cidx * _L, _L)] = (an - tc_v[pl.ds(cidx * _L, _L)]) / aw
        return c

    lax.fori_loop(0, nsys // _L, fbody, 0)

    def body(i, c):
        for j in range(_UNROLL):
            sl = pl.ds((i * _UNROLL + j) * _L, _L)
            fg = plsc.load_gather(f_v, [bi_v[sl]])
            q_v[sl] = nel_v[sl] - wi_v[sl] * fg
        return c

    lax.fori_loop(0, ch // (_L * _UNROLL), body, 0)
    pltpu.sync_copy(q_v, q_hbm.at[pl.ds(base, ch)])


def kernel(species, embedding, batch_index, natoms, total_charge, W, b):
    n = embedding.shape[0]
    nsys = natoms.shape[0]
    ch = n // _NW
    wi = _compute_wi(embedding, W, b)
    tbl = jnp.asarray(_VALENCE_TABLE)
    mesh = plsc.VectorSubcoreMesh(core_axis_name="c", subcore_axis_name="s")
    sc_params = pltpu.CompilerParams(
        needs_layout_passes=False, skip_device_barrier=True
    )

    def partials_body(*refs):
        _sc_partials_body(ch, nsys, *refs)

    nel, wparts, nparts = pl.kernel(
        partials_body,
        out_type=[
            jax.ShapeDtypeStruct((n,), jnp.float32),
            jax.ShapeDtypeStruct((_NW * nsys,), jnp.float32),
            jax.ShapeDtypeStruct((_NW * nsys,), jnp.float32),
        ],
        mesh=mesh,
        compiler_params=sc_params,
        scratch_types=[
            pltpu.VMEM((ch,), jnp.int32),
            pltpu.VMEM((ch,), jnp.int32),
            pltpu.VMEM((ch,), jnp.float32),
            pltpu.VMEM((ch,), jnp.float32),
            pltpu.VMEM((128,), jnp.float32),
            pltpu.VMEM((nsys,), jnp.float32),
            pltpu.VMEM((nsys,), jnp.float32),
            pltpu.SemaphoreType.DMA,
        ],
    )(species, batch_index, wi, tbl)

    def final_body(*refs):
        _sc_final_body(ch, nsys, *refs)

    q = pl.kernel(
        final_body,
        out_type=jax.ShapeDtypeStruct((n,), jnp.float32),
        mesh=mesh,
        compiler_params=sc_params,
        scratch_types=[
            pltpu.VMEM((_NW * nsys,), jnp.float32),
            pltpu.VMEM((_NW * nsys,), jnp.float32),
            pltpu.VMEM((nsys,), jnp.float32),
            pltpu.VMEM((nsys,), jnp.float32),
            pltpu.VMEM((ch,), jnp.float32),
            pltpu.VMEM((ch,), jnp.float32),
            pltpu.VMEM((ch,), jnp.int32),
            pltpu.VMEM((ch,), jnp.float32),
            pltpu.SemaphoreType.DMA,
        ],
    )(wparts, nparts, total_charge, wi, nel, batch_index)
    return q


# speedup vs baseline: 1.9159x; 1.0421x over previous
import jax
import jax.numpy as jnp
import numpy as np
from jax import lax
from jax.experimental import pallas as pl
from jax.experimental.pallas import tpu as pltpu
from jax.experimental.pallas import tpu_sc as plsc

_NOBLE_CORES = np.array([0, 2, 10, 18, 36, 54, 86, 118])


def _valence_count(z: int) -> float:
    if z <= 0:
        return 0.0
    return float(z - _NOBLE_CORES[_NOBLE_CORES < z].max())


_VALENCE_TABLE = np.zeros((128,), dtype=np.float32)
for _z in range(119):
    _VALENCE_TABLE[_z] = _valence_count(_z)

_BLK = 32000
_UNROLL = 25
_L = 16
_NC = 2
_NS = 16
_NW = _NC * _NS


def _wi_body(wt_ref, b_ref, emb_ref, wi_ref, buf_ref, sem):
    ei = lax.dot_general(
        wt_ref[...], emb_ref[...],
        (((1,), (1,)), ((), ())),
        preferred_element_type=jnp.float32,
    )
    x = ei + b_ref[0, 0]
    wi = jnp.maximum(x, 0.0) + jnp.log1p(jnp.exp(-jnp.abs(x)))
    buf_ref[...] = wi.reshape(_BLK)
    i = pl.program_id(0)
    cp = pltpu.make_async_copy(buf_ref, wi_ref.at[pl.ds(i * _BLK, _BLK)], sem)
    cp.start()
    cp.wait()


def _compute_wi(embedding, W, b):
    n, d = embedding.shape
    nb = n // _BLK
    wt = W.reshape(1, d)
    b2 = b.reshape(1, 1)
    wi = pl.pallas_call(
        _wi_body,
        grid=(nb,),
        in_specs=[
            pl.BlockSpec((1, d), lambda i: (0, 0)),
            pl.BlockSpec((1, 1), lambda i: (0, 0)),
            pl.BlockSpec((_BLK, d), lambda i: (i, 0)),
        ],
        out_specs=pl.BlockSpec(memory_space=pl.ANY),
        out_shape=jax.ShapeDtypeStruct((n,), jnp.float32),
        scratch_shapes=[
            pltpu.VMEM((_BLK,), jnp.float32),
            pltpu.SemaphoreType.DMA,
        ],
    )(wt, b2, embedding)
    return wi


def _sc_partials_body(ch, nsys, species_hbm, bi_hbm, wi_hbm, tbl_hbm,
                      nel_hbm, wparts_hbm, nparts_hbm,
                      spec_v, bi_v, wi_v, nel_v, tbl_v, wacc, nacc, sem):
    wid = lax.axis_index("s") * _NC + lax.axis_index("c")
    base = wid * ch
    sl_h = pl.ds(base, ch)
    copies = [
        pltpu.make_async_copy(tbl_hbm, tbl_v, sem),
        pltpu.make_async_copy(species_hbm.at[sl_h], spec_v, sem),
        pltpu.make_async_copy(bi_hbm.at[sl_h], bi_v, sem),
        pltpu.make_async_copy(wi_hbm.at[sl_h], wi_v, sem),
    ]
    for c in copies:
        c.start()

    zeros = jnp.zeros((_L,), jnp.float32)
    for j in range(nsys // _L):
        wacc[pl.ds(j * _L, _L)] = zeros
        nacc[pl.ds(j * _L, _L)] = zeros
    for c in copies:
        c.wait()

    lane_off = lax.iota(jnp.int32, _L) * (ch // _L)

    def body(i, c):
        for j in range(_UNROLL):
            idx = lane_off + i * _UNROLL + j
            sp = plsc.load_gather(spec_v, [idx])
            b16 = plsc.load_gather(bi_v, [idx])
            w16 = plsc.load_gather(wi_v, [idx])
            n16 = plsc.load_gather(tbl_v, [sp])
            plsc.store_scatter(nel_v, [idx], n16)
            plsc.addupdate_scatter(wacc, [b16], w16)
            plsc.addupdate_scatter(nacc, [b16], n16)
        return c

    lax.fori_loop(0, ch // (_L * _UNROLL), body, 0)

    pltpu.sync_copy(nel_v, nel_hbm.at[pl.ds(base, ch)])
    pltpu.sync_copy(wacc, wparts_hbm.at[pl.ds(wid * nsys, nsys)])
    pltpu.sync_copy(nacc, nparts_hbm.at[pl.ds(wid * nsys, nsys)])


def _sc_final_body(ch, nsys, wparts_hbm, nparts_hbm, tc_hbm,
                   wi_hbm, nel_hbm, bi_hbm, q_hbm,
                   wp_v, np_v, tc_v, f_v, wi_v, nel_v, bi_v, q_v, sem, sem2):
    wid = lax.axis_index("s") * _NC + lax.axis_index("c")
    base = wid * ch
    sl_h = pl.ds(base, ch)
    part_copies = [
        pltpu.make_async_copy(wparts_hbm, wp_v, sem),
        pltpu.make_async_copy(nparts_hbm, np_v, sem),
        pltpu.make_async_copy(tc_hbm, tc_v, sem),
    ]
    chunk_copies = [
        pltpu.make_async_copy(wi_hbm.at[sl_h], wi_v, sem2),
        pltpu.make_async_copy(nel_hbm.at[sl_h], nel_v, sem2),
        pltpu.make_async_copy(bi_hbm.at[sl_h], bi_v, sem2),
    ]
    for c in part_copies + chunk_copies:
        c.start()
    for c in part_copies:
        c.wait()

    zeros = jnp.zeros((_L,), jnp.float32)

    def fbody(cidx, c):
        aw, an = zeros, zeros
        for r in range(_NW):
            off = r * nsys + cidx * _L
            aw = aw + wp_v[pl.ds(off, _L)]
            an = an + np_v[pl.ds(off, _L)]
        f_v[pl.ds(cidx * _L, _L)] = (an - tc_v[pl.ds(cidx * _L, _L)]) / aw
        return c

    lax.fori_loop(0, nsys // _L, fbody, 0)
    for c in chunk_copies:
        c.wait()

    def body(i, c):
        for j in range(_UNROLL):
            sl = pl.ds((i * _UNROLL + j) * _L, _L)
            fg = plsc.load_gather(f_v, [bi_v[sl]])
            q_v[sl] = nel_v[sl] - wi_v[sl] * fg
        return c

    lax.fori_loop(0, ch // (_L * _UNROLL), body, 0)
    pltpu.sync_copy(q_v, q_hbm.at[pl.ds(base, ch)])


def kernel(species, embedding, batch_index, natoms, total_charge, W, b):
    n = embedding.shape[0]
    nsys = natoms.shape[0]
    ch = n // _NW
    wi = _compute_wi(embedding, W, b)
    tbl = jnp.asarray(_VALENCE_TABLE)
    mesh = plsc.VectorSubcoreMesh(core_axis_name="c", subcore_axis_name="s")
    sc_params = pltpu.CompilerParams(
        needs_layout_passes=False, skip_device_barrier=True
    )

    def partials_body(*refs):
        _sc_partials_body(ch, nsys, *refs)

    nel, wparts, nparts = pl.kernel(
        partials_body,
        out_type=[
            jax.ShapeDtypeStruct((n,), jnp.float32),
            jax.ShapeDtypeStruct((_NW * nsys,), jnp.float32),
            jax.ShapeDtypeStruct((_NW * nsys,), jnp.float32),
        ],
        mesh=mesh,
        compiler_params=sc_params,
        scratch_types=[
            pltpu.VMEM((ch,), jnp.int32),
            pltpu.VMEM((ch,), jnp.int32),
            pltpu.VMEM((ch,), jnp.float32),
            pltpu.VMEM((ch,), jnp.float32),
            pltpu.VMEM((128,), jnp.float32),
            pltpu.VMEM((nsys,), jnp.float32),
            pltpu.VMEM((nsys,), jnp.float32),
            pltpu.SemaphoreType.DMA,
        ],
    )(species, batch_index, wi, tbl)

    def final_body(*refs):
        _sc_final_body(ch, nsys, *refs)

    q = pl.kernel(
        final_body,
        out_type=jax.ShapeDtypeStruct((n,), jnp.float32),
        mesh=mesh,
        compiler_params=sc_params,
        scratch_types=[
            pltpu.VMEM((_NW * nsys,), jnp.float32),
            pltpu.VMEM((_NW * nsys,), jnp.float32),
            pltpu.VMEM((nsys,), jnp.float32),
            pltpu.VMEM((nsys,), jnp.float32),
            pltpu.VMEM((ch,), jnp.float32),
            pltpu.VMEM((ch,), jnp.float32),
            pltpu.VMEM((ch,), jnp.int32),
            pltpu.VMEM((ch,), jnp.float32),
            pltpu.SemaphoreType.DMA,
            pltpu.SemaphoreType.DMA,
        ],
    )(wparts, nparts, total_charge, wi, nel, batch_index)
    return q


# speedup vs baseline: 1.9644x; 1.0253x over previous
import jax
import jax.numpy as jnp
import numpy as np
from jax import lax
from jax.experimental import pallas as pl
from jax.experimental.pallas import tpu as pltpu
from jax.experimental.pallas import tpu_sc as plsc

_NOBLE_CORES = np.array([0, 2, 10, 18, 36, 54, 86, 118])


def _valence_count(z: int) -> float:
    if z <= 0:
        return 0.0
    return float(z - _NOBLE_CORES[_NOBLE_CORES < z].max())


_VALENCE_TABLE = np.zeros((128,), dtype=np.float32)
for _z in range(119):
    _VALENCE_TABLE[_z] = _valence_count(_z)

_BLK = 32000
_UNROLL = 5
_L = 16
_NC = 2
_NS = 16
_NW = _NC * _NS


def _wi_body(wt_ref, b_ref, emb_ref, wi_ref, buf_ref, sem):
    ei = lax.dot_general(
        wt_ref[...], emb_ref[...],
        (((1,), (1,)), ((), ())),
        preferred_element_type=jnp.float32,
    )
    x = ei + b_ref[0, 0]
    wi = jnp.maximum(x, 0.0) + jnp.log1p(jnp.exp(-jnp.abs(x)))
    buf_ref[...] = wi.reshape(_BLK)
    i = pl.program_id(0)
    cp = pltpu.make_async_copy(buf_ref, wi_ref.at[pl.ds(i * _BLK, _BLK)], sem)
    cp.start()
    cp.wait()


def _compute_wi(embedding, W, b):
    n, d = embedding.shape
    nb = n // _BLK
    wt = W.reshape(1, d)
    b2 = b.reshape(1, 1)
    wi = pl.pallas_call(
        _wi_body,
        grid=(nb,),
        in_specs=[
            pl.BlockSpec((1, d), lambda i: (0, 0)),
            pl.BlockSpec((1, 1), lambda i: (0, 0)),
            pl.BlockSpec((_BLK, d), lambda i: (i, 0)),
        ],
        out_specs=pl.BlockSpec(memory_space=pl.ANY),
        out_shape=jax.ShapeDtypeStruct((n,), jnp.float32),
        scratch_shapes=[
            pltpu.VMEM((_BLK,), jnp.float32),
            pltpu.SemaphoreType.DMA,
        ],
    )(wt, b2, embedding)
    return wi


def _sc_partials_body(ch, nsys, species_hbm, bi_hbm, wi_hbm, tbl_hbm,
                      nel_hbm, wparts_hbm, nparts_hbm,
                      spec_v, bi_v, wi_v, nel_v, tbl_v, wacc, nacc, sem):
    wid = lax.axis_index("s") * _NC + lax.axis_index("c")
    base = wid * ch
    sl_h = pl.ds(base, ch)
    copies = [
        pltpu.make_async_copy(tbl_hbm, tbl_v, sem),
        pltpu.make_async_copy(species_hbm.at[sl_h], spec_v, sem),
        pltpu.make_async_copy(bi_hbm.at[sl_h], bi_v, sem),
        pltpu.make_async_copy(wi_hbm.at[sl_h], wi_v, sem),
    ]
    for c in copies:
        c.start()

    zeros = jnp.zeros((_L,), jnp.float32)
    for j in range(nsys // _L):
        wacc[pl.ds(j * _L, _L)] = zeros
        nacc[pl.ds(j * _L, _L)] = zeros
    for c in copies:
        c.wait()

    lane_off = lax.iota(jnp.int32, _L) * (ch // _L)

    def body(i, c):
        for j in range(_UNROLL):
            idx = lane_off + i * _UNROLL + j
            sp = plsc.load_gather(spec_v, [idx])
            b16 = plsc.load_gather(bi_v, [idx])
            w16 = plsc.load_gather(wi_v, [idx])
            n16 = plsc.load_gather(tbl_v, [sp])
            plsc.store_scatter(nel_v, [idx], n16)
            plsc.addupdate_scatter(wacc, [b16], w16)
            plsc.addupdate_scatter(nacc, [b16], n16)
        return c

    lax.fori_loop(0, ch // (_L * _UNROLL), body, 0)

    pltpu.sync_copy(nel_v, nel_hbm.at[pl.ds(base, ch)])
    pltpu.sync_copy(wacc, wparts_hbm.at[pl.ds(wid * nsys, nsys)])
    pltpu.sync_copy(nacc, nparts_hbm.at[pl.ds(wid * nsys, nsys)])


def _sc_final_body(ch, nsys, wparts_hbm, nparts_hbm, tc_hbm,
                   wi_hbm, nel_hbm, bi_hbm, q_hbm,
                   wp_v, np_v, tc_v, f_v, wi_v, nel_v, bi_v, q_v, sem, sem2):
    wid = lax.axis_index("s") * _NC + lax.axis_index("c")
    base = wid * ch
    sl_h = pl.ds(base, ch)
    part_copies = [
        pltpu.make_async_copy(wparts_hbm, wp_v, sem),
        pltpu.make_async_copy(nparts_hbm, np_v, sem),
        pltpu.make_async_copy(tc_hbm, tc_v, sem),
    ]
    chunk_copies = [
        pltpu.make_async_copy(wi_hbm.at[sl_h], wi_v, sem2),
        pltpu.make_async_copy(nel_hbm.at[sl_h], nel_v, sem2),
        pltpu.make_async_copy(bi_hbm.at[sl_h], bi_v, sem2),
    ]
    for c in part_copies + chunk_copies:
        c.start()
    for c in part_copies:
        c.wait()

    zeros = jnp.zeros((_L,), jnp.float32)

    def fbody(cidx, c):
        aw, an = zeros, zeros
        for r in range(_NW):
            off = r * nsys + cidx * _L
            aw = aw + wp_v[pl.ds(off, _L)]
            an = an + np_v[pl.ds(off, _L)]
        f_v[pl.ds(cidx * _L, _L)] = (an - tc_v[pl.ds(cidx * _L, _L)]) / aw
        return c

    lax.fori_loop(0, nsys // _L, fbody, 0)
    for c in chunk_copies:
        c.wait()

    def body(i, c):
        for j in range(_UNROLL):
            sl = pl.ds((i * _UNROLL + j) * _L, _L)
            fg = plsc.load_gather(f_v, [bi_v[sl]])
            q_v[sl] = nel_v[sl] - wi_v[sl] * fg
        return c

    lax.fori_loop(0, ch // (_L * _UNROLL), body, 0)
    pltpu.sync_copy(q_v, q_hbm.at[pl.ds(base, ch)])


def kernel(species, embedding, batch_index, natoms, total_charge, W, b):
    n = embedding.shape[0]
    nsys = natoms.shape[0]
    ch = n // _NW
    wi = _compute_wi(embedding, W, b)
    tbl = jnp.asarray(_VALENCE_TABLE)
    mesh = plsc.VectorSubcoreMesh(core_axis_name="c", subcore_axis_name="s")
    sc_params = pltpu.CompilerParams(
        needs_layout_passes=False, skip_device_barrier=True
    )

    def partials_body(*refs):
        _sc_partials_body(ch, nsys, *refs)

    nel, wparts, nparts = pl.kernel(
        partials_body,
        out_type=[
            jax.ShapeDtypeStruct((n,), jnp.float32),
            jax.ShapeDtypeStruct((_NW * nsys,), jnp.float32),
            jax.ShapeDtypeStruct((_NW * nsys,), jnp.float32),
        ],
        mesh=mesh,
        compiler_params=sc_params,
        scratch_types=[
            pltpu.VMEM((ch,), jnp.int32),
            pltpu.VMEM((ch,), jnp.int32),
            pltpu.VMEM((ch,), jnp.float32),
            pltpu.VMEM((ch,), jnp.float32),
            pltpu.VMEM((128,), jnp.float32),
            pltpu.VMEM((nsys,), jnp.float32),
            pltpu.VMEM((nsys,), jnp.float32),
            pltpu.SemaphoreType.DMA,
        ],
    )(species, batch_index, wi, tbl)

    def final_body(*refs):
        _sc_final_body(ch, nsys, *refs)

    q = pl.kernel(
        final_body,
        out_type=jax.ShapeDtypeStruct((n,), jnp.float32),
        mesh=mesh,
        compiler_params=sc_params,
        scratch_types=[
            pltpu.VMEM((_NW * nsys,), jnp.float32),
            pltpu.VMEM((_NW * nsys,), jnp.float32),
            pltpu.VMEM((nsys,), jnp.float32),
            pltpu.VMEM((nsys,), jnp.float32),
            pltpu.VMEM((ch,), jnp.float32),
            pltpu.VMEM((ch,), jnp.float32),
            pltpu.VMEM((ch,), jnp.int32),
            pltpu.VMEM((ch,), jnp.float32),
            pltpu.SemaphoreType.DMA,
            pltpu.SemaphoreType.DMA,
        ],
    )(wparts, nparts, total_charge, wi, nel, batch_index)
    return q
